# baseline (speedup 1.0000x reference)
import functools

import jax
import jax.numpy as jnp
from jax import lax
from jax.experimental import pallas as pl
from jax.experimental.pallas import tpu as pltpu
from jax.experimental.pallas import tpu_sc as plsc


@functools.lru_cache(maxsize=None)
def _make_gather(V, D, B):
    info = plsc.get_sparse_core_info()
    NC, NS = info.num_cores, info.num_subcores
    NW = NC * NS
    assert B % NW == 0
    b_per_w = B // NW
    mesh = plsc.VectorSubcoreMesh(core_axis_name="c", subcore_axis_name="s")

    @functools.partial(
        pl.kernel,
        mesh=mesh,
        compiler_params=pltpu.CompilerParams(use_tc_tiling_on_sc=False),
        out_type=jax.ShapeDtypeStruct((B, D), jnp.float32),
        scratch_types=[
            pltpu.VMEM((b_per_w,), jnp.int32),
            pltpu.VMEM((b_per_w, D), jnp.float32),
            pltpu.SemaphoreType.DMA,
        ],
    )
    def k(idx_hbm, table_hbm, out_hbm, idx_v, rows_v, sem):
        wid = lax.axis_index("s") * NC + lax.axis_index("c")
        base = wid * b_per_w
        pltpu.sync_copy(idx_hbm.at[pl.ds(base, b_per_w)], idx_v)
        pltpu.async_copy(table_hbm.at[idx_v], rows_v, sem).wait()
        pltpu.sync_copy(rows_v, out_hbm.at[pl.ds(base, b_per_w)])

    return k


def kernel(indices, table):
    idx = indices.astype(jnp.int32)
    (B,) = idx.shape
    V, D = table.shape
    return _make_gather(V, D, B)(idx, table)


# speedup vs baseline: 1.7298x; 1.7298x over previous
import functools

import jax
import jax.numpy as jnp
from jax import lax
from jax.experimental import pallas as pl
from jax.experimental.pallas import tpu as pltpu
from jax.experimental.pallas import tpu_sc as plsc


@functools.lru_cache(maxsize=None)
def _make_gather(V, D, B):
    info = plsc.get_sparse_core_info()
    NC, NS = info.num_cores, info.num_subcores
    NW = NC * NS
    assert B % NW == 0
    b_per_w = B // NW
    mesh = plsc.VectorSubcoreMesh(core_axis_name="c", subcore_axis_name="s")

    @functools.partial(
        pl.kernel,
        mesh=mesh,
        out_type=jax.ShapeDtypeStruct((B, D), jnp.float32),
        scratch_types=[
            pltpu.VMEM((b_per_w,), jnp.int32),
            pltpu.VMEM((b_per_w, D), jnp.float32),
            pltpu.SemaphoreType.DMA,
        ],
    )
    def k(idx_hbm, table_hbm, out_hbm, idx_v, rows_v, sem):
        wid = lax.axis_index("s") * NC + lax.axis_index("c")
        base = wid * b_per_w
        pltpu.sync_copy(idx_hbm.at[pl.ds(base, b_per_w)], idx_v)

        def body(c, _):
            b0 = c * 16
            vec = idx_v[pl.ds(b0, 16)]
            for t in range(16):
                pltpu.async_copy(table_hbm.at[vec[t]], rows_v.at[b0 + t], sem)
            return ()

        lax.fori_loop(0, b_per_w // 16, body, ())
        pltpu.make_async_copy(table_hbm.at[pl.ds(0, b_per_w)], rows_v, sem).wait()
        pltpu.sync_copy(rows_v, out_hbm.at[pl.ds(base, b_per_w)])

    return k


def kernel(indices, table):
    idx = indices.astype(jnp.int32)
    (B,) = idx.shape
    V, D = table.shape
    return _make_gather(V, D, B)(idx, table)


# speedup vs baseline: 1.8797x; 1.0866x over previous
import functools

import jax
import jax.numpy as jnp
from jax import lax
from jax.experimental import pallas as pl
from jax.experimental.pallas import tpu as pltpu
from jax.experimental.pallas import tpu_sc as plsc


@functools.lru_cache(maxsize=None)
def _make_gather(V, D, B):
    info = plsc.get_sparse_core_info()
    NC, NS, L = info.num_cores, info.num_subcores, info.num_lanes
    NW = NC * NS
    assert B % (16 * NW) == 0 and D % L == 0
    b_per_w = B // NW
    n_grp = b_per_w // 16
    mesh = plsc.VectorSubcoreMesh(core_axis_name="c", subcore_axis_name="s")

    @functools.partial(
        pl.kernel,
        mesh=mesh,
        compiler_params=pltpu.CompilerParams(needs_layout_passes=False),
        out_type=jax.ShapeDtypeStruct((B, D), jnp.float32),
        scratch_types=[
            pltpu.VMEM((b_per_w,), jnp.int32),
            pltpu.VMEM((D, 128), jnp.float32),
            pltpu.VMEM((D, 128), jnp.float32),
            pltpu.VMEM((b_per_w, D), jnp.float32),
            pltpu.SemaphoreType.DMA,
            pltpu.SemaphoreType.DMA,
        ],
    )
    def k(idx_hbm, tableT_hbm, out_hbm, idx_v, chunk0, chunk1, rows_v, s0, s1):
        wid = lax.axis_index("s") * NC + lax.axis_index("c")
        base = wid * b_per_w
        pltpu.sync_copy(idx_hbm.at[pl.ds(base, b_per_w)], idx_v)

        chunks = (chunk0, chunk1)
        sems = (s0, s1)
        iota = lax.iota(jnp.int32, L)

        def fire(i, slot):
            cc = pl.multiple_of((i >> 7) * 128, 128)
            pltpu.async_copy(
                tableT_hbm.at[:, pl.ds(cc, 128)], chunks[slot], sems[slot]
            )

        def wait(slot):
            pltpu.make_async_copy(
                tableT_hbm.at[:, pl.ds(0, 128)], chunks[slot], sems[slot]
            ).wait()

        def extract(i, slot, n):
            lane = jnp.broadcast_to(i & 127, (L,))
            for g2 in range(D // L):
                v = plsc.load_gather(chunks[slot], [iota + L * g2, lane])
                rows_v[n, pl.ds(L * g2, L)] = v

        first = idx_v[pl.ds(0, 16)]
        fire(first[0], 0)

        def body(g, _):
            vec = idx_v[pl.ds(g * 16, 16)]
            nxt = idx_v[pl.ds(jnp.minimum(g + 1, n_grp - 1) * 16, 16)]
            for t in range(16):
                n = g * 16 + t
                slot = t % 2
                if t < 15:
                    fire(vec[t + 1], (t + 1) % 2)
                else:

                    @pl.when(g < n_grp - 1)
                    def _():
                        fire(nxt[0], 0)

                wait(slot)
                extract(vec[t], slot, n)
            return ()

        lax.fori_loop(0, n_grp, body, ())
        pltpu.sync_copy(rows_v, out_hbm.at[pl.ds(base, b_per_w)])

    return k


def kernel(indices, table):
    idx = indices.astype(jnp.int32)
    (B,) = idx.shape
    V, D = table.shape
    return _make_gather(V, D, B)(idx, table.T)


# speedup vs baseline: 2.5233x; 1.3424x over previous
import functools

import jax
import jax.numpy as jnp
from jax import lax
from jax.experimental import pallas as pl
from jax.experimental.pallas import tpu as pltpu
from jax.experimental.pallas import tpu_sc as plsc


@functools.lru_cache(maxsize=None)
def _make_gather(V, D, B):
    info = plsc.get_sparse_core_info()
    NC, NS, L = info.num_cores, info.num_subcores, info.num_lanes
    NW = NC * NS
    assert B % (16 * NW) == 0 and D % L == 0
    b_per_w = B // NW
    n_grp = b_per_w // 16
    mesh = plsc.VectorSubcoreMesh(core_axis_name="c", subcore_axis_name="s")

    @functools.partial(
        pl.kernel,
        mesh=mesh,
        compiler_params=pltpu.CompilerParams(needs_layout_passes=False),
        out_type=jax.ShapeDtypeStruct((B, D), jnp.float32),
        scratch_types=[
            pltpu.VMEM((b_per_w,), jnp.int32),
            pltpu.VMEM((D, 128), jnp.float32),
            pltpu.VMEM((D, 128), jnp.float32),
            pltpu.VMEM((D, 128), jnp.float32),
            pltpu.VMEM((D, 128), jnp.float32),
            pltpu.VMEM((b_per_w, D), jnp.float32),
            pltpu.SemaphoreType.DMA,
            pltpu.SemaphoreType.DMA,
            pltpu.SemaphoreType.DMA,
            pltpu.SemaphoreType.DMA,
        ],
    )
    def k(idx_hbm, tableT_hbm, out_hbm, idx_v, c0, c1, c2, c3, rows_v,
          s0, s1, s2, s3):
        wid = lax.axis_index("s") * NC + lax.axis_index("c")
        base = wid * b_per_w
        pltpu.sync_copy(idx_hbm.at[pl.ds(base, b_per_w)], idx_v)

        chunks = (c0, c1, c2, c3)
        sems = (s0, s1, s2, s3)
        iota = lax.iota(jnp.int32, L)

        def fire(i, slot):
            cc = pl.multiple_of((i >> 7) * 128, 128)
            pltpu.async_copy(
                tableT_hbm.at[:, pl.ds(cc, 128)], chunks[slot], sems[slot]
            )

        def wait(slot):
            pltpu.make_async_copy(
                tableT_hbm.at[:, pl.ds(0, 128)], chunks[slot], sems[slot]
            ).wait()

        def extract(i, slot, n):
            lane = jnp.broadcast_to(i & 127, (L,))
            for g2 in range(D // L):
                v = plsc.load_gather(chunks[slot], [iota + L * g2, lane])
                rows_v[n, pl.ds(L * g2, L)] = v

        first = idx_v[pl.ds(0, 16)]
        for p in range(3):
            fire(first[p], p)

        def body(g, _):
            vec = idx_v[pl.ds(g * 16, 16)]
            nxt = idx_v[pl.ds(jnp.minimum(g + 1, n_grp - 1) * 16, 16)]
            for t in range(16):
                n = g * 16 + t
                slot = t % 4
                if t < 13:
                    fire(vec[t + 3], (t + 3) % 4)
                else:

                    @pl.when(g < n_grp - 1)
                    def _():
                        fire(nxt[t - 13], (t + 3) % 4)

                wait(slot)
                extract(vec[t], slot, n)
            return ()

        lax.fori_loop(0, n_grp, body, ())
        pltpu.sync_copy(rows_v, out_hbm.at[pl.ds(base, b_per_w)])

    return k


def kernel(indices, table):
    idx = indices.astype(jnp.int32)
    (B,) = idx.shape
    V, D = table.shape
    return _make_gather(V, D, B)(idx, table.T)


# speedup vs baseline: 2.8919x; 1.1461x over previous
import functools

import jax
import jax.numpy as jnp
from jax import lax
from jax.experimental import pallas as pl
from jax.experimental.pallas import tpu as pltpu
from jax.experimental.pallas import tpu_sc as plsc

_DEPTH = 8


@functools.lru_cache(maxsize=None)
def _make_gather(V, D, B):
    info = plsc.get_sparse_core_info()
    NC, NS, L = info.num_cores, info.num_subcores, info.num_lanes
    NW = NC * NS
    assert B % (16 * NW) == 0 and D % L == 0 and 16 % _DEPTH == 0
    b_per_w = B // NW
    n_grp = b_per_w // 16
    ahead = _DEPTH - 1
    mesh = plsc.VectorSubcoreMesh(core_axis_name="c", subcore_axis_name="s")

    @functools.partial(
        pl.kernel,
        mesh=mesh,
        compiler_params=pltpu.CompilerParams(needs_layout_passes=False),
        out_type=jax.ShapeDtypeStruct((B * D,), jnp.float32),
        scratch_types=[
            pltpu.VMEM((b_per_w,), jnp.int32),
            *[pltpu.VMEM((D, 128), jnp.float32) for _ in range(_DEPTH)],
            pltpu.VMEM((b_per_w * D,), jnp.float32),
            *[pltpu.SemaphoreType.DMA for _ in range(_DEPTH)],
        ],
    )
    def k(idx_hbm, tableT_hbm, out_hbm, idx_v, *rest):
        chunks = rest[:_DEPTH]
        rows_v = rest[_DEPTH]
        sems = rest[_DEPTH + 1 :]
        wid = lax.axis_index("s") * NC + lax.axis_index("c")
        base = wid * b_per_w
        pltpu.sync_copy(idx_hbm.at[pl.ds(base, b_per_w)], idx_v)

        iota = lax.iota(jnp.int32, L)

        def fire(i, slot):
            cc = pl.multiple_of((i >> 7) * 128, 128)
            pltpu.async_copy(
                tableT_hbm.at[:, pl.ds(cc, 128)], chunks[slot], sems[slot]
            )

        def wait(slot):
            pltpu.make_async_copy(
                tableT_hbm.at[:, pl.ds(0, 128)], chunks[slot], sems[slot]
            ).wait()

        def extract(i, slot, n):
            lane = jnp.broadcast_to(i & 127, (L,))
            for g2 in range(D // L):
                v = plsc.load_gather(chunks[slot], [iota + L * g2, lane])
                rows_v[pl.ds(n * D + L * g2, L)] = v

        first = idx_v[pl.ds(0, 16)]
        for p in range(ahead):
            fire(first[p], p)

        def body(g, _):
            vec = idx_v[pl.ds(g * 16, 16)]
            nxt = idx_v[pl.ds(jnp.minimum(g + 1, n_grp - 1) * 16, 16)]
            for t in range(16):
                n = g * 16 + t
                slot = t % _DEPTH
                if t < 16 - ahead:
                    fire(vec[t + ahead], (t + ahead) % _DEPTH)
                else:

                    @pl.when(g < n_grp - 1)
                    def _():
                        fire(nxt[t + ahead - 16], (t + ahead) % _DEPTH)

                wait(slot)
                extract(vec[t], slot, n)
            return ()

        lax.fori_loop(0, n_grp, body, ())
        pltpu.sync_copy(rows_v, out_hbm.at[pl.ds(base * D, b_per_w * D)])

    return k


def kernel(indices, table):
    idx = indices.astype(jnp.int32)
    (B,) = idx.shape
    V, D = table.shape
    flat = _make_gather(V, D, B)(idx, table.T)
    return flat.reshape(B, D)


# speedup vs baseline: 3.7068x; 1.2818x over previous
import functools

import jax
import jax.numpy as jnp
from jax import lax
from jax.experimental import pallas as pl
from jax.experimental.pallas import tpu as pltpu
from jax.experimental.pallas import tpu_sc as plsc

_DEPTH = 4
_WLMAX = 1024
_ROWS = 256


@functools.lru_cache(maxsize=None)
def _make_gather(V, D, B):
    info = plsc.get_sparse_core_info()
    NC, NS, L = info.num_cores, info.num_subcores, info.num_lanes
    NW = NC * NS
    assert B % (16 * NW) == 0 and D % L == 0
    n_chunk = (V + 127) // 128
    cpw = (n_chunk + NW - 1) // NW
    n_hist = ((cpw + 15) // 16) * 16
    mesh = plsc.VectorSubcoreMesh(core_axis_name="c", subcore_axis_name="s")

    @functools.partial(
        pl.kernel,
        mesh=mesh,
        compiler_params=pltpu.CompilerParams(needs_layout_passes=False),
        out_type=jax.ShapeDtypeStruct((B, D), jnp.float32),
        scratch_types=[
            pltpu.VMEM((B,), jnp.int32),
            pltpu.VMEM((_WLMAX + 16,), jnp.int32),
            pltpu.VMEM((_WLMAX + 16,), jnp.int32),
            pltpu.VMEM((_WLMAX + 16,), jnp.int32),
            pltpu.VMEM((_WLMAX + 16,), jnp.int32),
            pltpu.VMEM((n_hist + 16,), jnp.int32),
            pltpu.VMEM((n_hist + 16,), jnp.int32),
            pltpu.VMEM((n_hist + 16,), jnp.int32),
            *[pltpu.VMEM((D, 128), jnp.float32) for _ in range(_DEPTH)],
            pltpu.VMEM((_ROWS, D), jnp.float32),
            pltpu.SemaphoreType.DMA,
            *[pltpu.SemaphoreType.DMA for _ in range(_DEPTH)],
        ],
    )
    def k(idx_hbm, tableT_hbm, out_hbm, idx_v, wl_i, wl_j, ws_i, ws_j,
          hist, starts0, cursor, *rest):
        chunks = rest[:_DEPTH]
        rows_v = rest[_DEPTH]
        sem_out = rest[_DEPTH + 1]
        sems = rest[_DEPTH + 2 :]
        wid = lax.axis_index("s") * NC + lax.axis_index("c")
        c_lo = wid * cpw
        c_hi = jnp.minimum(c_lo + cpw, n_chunk)
        nc_w = c_hi - c_lo

        pltpu.sync_copy(idx_hbm, idx_v)

        iota = lax.iota(jnp.int32, L)
        lane0 = iota == 0
        zeros = jnp.zeros((L,), jnp.int32)

        def scan_body(n, off):
            vec = idx_v[pl.ds(n * L, L)]
            cvec = vec >> 7
            mask = (cvec >= c_lo) & (cvec < c_hi)
            plsc.store_compressed(wl_i.at[pl.ds(off, L)], vec, mask=mask)
            plsc.store_compressed(
                wl_j.at[pl.ds(off, L)], n * L + iota, mask=mask
            )
            return off + plsc.all_reduce_population_count(mask)[0]

        n_wl = lax.fori_loop(0, B // L, scan_body, jnp.int32(0))

        for gp in range(n_hist // 16):
            hist[pl.ds(gp * 16, 16)] = zeros

        def count_body(q, _):
            iq = wl_i[pl.ds(q, L)][0]
            b = (iq >> 7) - c_lo
            cnt = hist[pl.ds(b, L)]
            plsc.store_compressed(hist.at[pl.ds(b, L)], cnt + 1, mask=lane0)
            return 0

        lax.fori_loop(0, n_wl, count_body, 0)

        carry = jnp.int32(0)
        for gp in range(n_hist // 16):
            hv = hist[pl.ds(gp * 16, 16)]
            cs = plsc.cumsum(hv)
            sv = cs - hv + carry
            starts0[pl.ds(gp * 16, 16)] = sv
            cursor[pl.ds(gp * 16, 16)] = sv
            carry = sv[15] + hv[15]

        def place_body(q, _):
            iq = wl_i[pl.ds(q, L)][0]
            jq = wl_j[pl.ds(q, L)][0]
            b = (iq >> 7) - c_lo
            pos = cursor[pl.ds(b, L)][0]
            plsc.store_compressed(
                cursor.at[pl.ds(b, L)],
                jnp.broadcast_to(pos + 1, (L,)),
                mask=lane0,
            )
            plsc.store_compressed(
                ws_i.at[pl.ds(pos, L)], jnp.broadcast_to(iq, (L,)), mask=lane0
            )
            plsc.store_compressed(
                ws_j.at[pl.ds(pos, L)], jnp.broadcast_to(jq, (L,)), mask=lane0
            )
            return 0

        lax.fori_loop(0, n_wl, place_body, 0)

        def fire(kk, slot):
            cc = pl.multiple_of((c_lo + kk) * 128, 128)
            pltpu.async_copy(
                tableT_hbm.at[:, pl.ds(cc, 128)], chunks[slot], sems[slot]
            )

        def wait(slot):
            pltpu.make_async_copy(
                tableT_hbm.at[:, pl.ds(0, 128)], chunks[slot], sems[slot]
            ).wait()

        for p in range(_DEPTH - 1):

            @pl.when(p < nc_w)
            def _():
                fire(p, p)

        def sweep_body(it, r):
            for m in range(_DEPTH):
                kk = it * _DEPTH + m
                valid = kk < nc_w

                @pl.when((kk + _DEPTH - 1) < nc_w)
                def _():
                    fire(kk + _DEPTH - 1, (m + _DEPTH - 1) % _DEPTH)

                @pl.when(valid)
                def _():
                    wait(m)

                lo = starts0[pl.ds(jnp.minimum(kk, n_hist - 1), L)][0]
                hi = cursor[pl.ds(jnp.minimum(kk, n_hist - 1), L)][0]
                lo = jnp.where(valid, lo, 0)
                hi = jnp.where(valid, hi, 0)

                def extract_body(q, rr):
                    @pl.when(rr >= _ROWS)
                    def _():
                        pltpu.make_async_copy(
                            rows_v.at[0], out_hbm.at[0], sem_out
                        ).wait()

                    slot_r = rr & (_ROWS - 1)
                    iq = ws_i[pl.ds(q, L)][0]
                    jq = ws_j[pl.ds(q, L)][0]
                    lane = jnp.broadcast_to(iq & 127, (L,))
                    for g2 in range(D // L):
                        v = plsc.load_gather(
                            chunks[m], [iota + L * g2, lane]
                        )
                        rows_v[slot_r, pl.ds(L * g2, L)] = v
                    pltpu.async_copy(
                        rows_v.at[slot_r], out_hbm.at[jq], sem_out
                    )
                    return rr + 1

                r = lax.fori_loop(lo, hi, extract_body, r)
            return r

        n_it = (cpw + _DEPTH - 1) // _DEPTH
        r_total = lax.fori_loop(0, n_it, sweep_body, jnp.int32(0))

        def drain_body(q, _):
            pltpu.make_async_copy(
                rows_v.at[0], out_hbm.at[0], sem_out
            ).wait()
            return 0

        lax.fori_loop(0, jnp.minimum(r_total, _ROWS), drain_body, 0)

    return k


def kernel(indices, table):
    idx = indices.astype(jnp.int32)
    (B,) = idx.shape
    V, D = table.shape
    return _make_gather(V, D, B)(idx, table.T)


# speedup vs baseline: 4.0584x; 1.0948x over previous
import functools

import jax
import jax.numpy as jnp
from jax import lax
from jax.experimental import pallas as pl
from jax.experimental.pallas import tpu as pltpu
from jax.experimental.pallas import tpu_sc as plsc

_DEPTH = 8
_WLMAX = 1024
_ROWS = 256


@functools.lru_cache(maxsize=None)
def _make_gather(V, D, B):
    info = plsc.get_sparse_core_info()
    NC, NS, L = info.num_cores, info.num_subcores, info.num_lanes
    NW = NC * NS
    assert B % (16 * NW) == 0 and D % L == 0
    n_chunk = (V + 127) // 128
    cpw = (n_chunk + NW - 1) // NW
    n_hist = ((cpw + 15) // 16) * 16
    mesh = plsc.VectorSubcoreMesh(core_axis_name="c", subcore_axis_name="s")

    @functools.partial(
        pl.kernel,
        mesh=mesh,
        compiler_params=pltpu.CompilerParams(needs_layout_passes=False),
        out_type=jax.ShapeDtypeStruct((B, D), jnp.float32),
        scratch_types=[
            pltpu.VMEM((B,), jnp.int32),
            pltpu.VMEM((_WLMAX + 16,), jnp.int32),
            pltpu.VMEM((_WLMAX + 16,), jnp.int32),
            pltpu.VMEM((_WLMAX + 16,), jnp.int32),
            pltpu.VMEM((_WLMAX + 16,), jnp.int32),
            pltpu.VMEM((n_hist + 16,), jnp.int32),
            pltpu.VMEM((n_hist + 16,), jnp.int32),
            pltpu.VMEM((n_hist + 16,), jnp.int32),
            *[pltpu.VMEM((D, 128), jnp.float32) for _ in range(_DEPTH)],
            pltpu.VMEM((_ROWS, D), jnp.float32),
            pltpu.SemaphoreType.DMA,
            *[pltpu.SemaphoreType.DMA for _ in range(_DEPTH)],
        ],
    )
    def k(idx_hbm, tableT_hbm, out_hbm, idx_v, wl_i, wl_j, ws_i, ws_j,
          hist, starts0, cursor, *rest):
        chunks = rest[:_DEPTH]
        rows_v = rest[_DEPTH]
        sem_out = rest[_DEPTH + 1]
        sems = rest[_DEPTH + 2 :]
        wid = lax.axis_index("s") * NC + lax.axis_index("c")
        c_lo = wid * cpw
        c_hi = jnp.minimum(c_lo + cpw, n_chunk)
        nc_w = c_hi - c_lo

        pltpu.sync_copy(idx_hbm, idx_v)

        iota = lax.iota(jnp.int32, L)
        lane0 = iota == 0
        zeros = jnp.zeros((L,), jnp.int32)

        def scan_body(n, off):
            vec = idx_v[pl.ds(n * L, L)]
            cvec = vec >> 7
            mask = (cvec >= c_lo) & (cvec < c_hi)
            plsc.store_compressed(wl_i.at[pl.ds(off, L)], vec, mask=mask)
            plsc.store_compressed(
                wl_j.at[pl.ds(off, L)], n * L + iota, mask=mask
            )
            return off + plsc.all_reduce_population_count(mask)[0]

        n_wl = lax.fori_loop(0, B // L, scan_body, jnp.int32(0))

        for gp in range(n_hist // 16):
            hist[pl.ds(gp * 16, 16)] = zeros

        def count_body(q, _):
            iq = wl_i[pl.ds(q, L)][0]
            b = (iq >> 7) - c_lo
            cnt = hist[pl.ds(b, L)]
            plsc.store_compressed(hist.at[pl.ds(b, L)], cnt + 1, mask=lane0)
            return 0

        lax.fori_loop(0, n_wl, count_body, 0)

        carry = jnp.int32(0)
        for gp in range(n_hist // 16):
            hv = hist[pl.ds(gp * 16, 16)]
            cs = plsc.cumsum(hv)
            sv = cs - hv + carry
            starts0[pl.ds(gp * 16, 16)] = sv
            cursor[pl.ds(gp * 16, 16)] = sv
            carry = sv[15] + hv[15]

        def place_body(q, _):
            iq = wl_i[pl.ds(q, L)][0]
            jq = wl_j[pl.ds(q, L)][0]
            b = (iq >> 7) - c_lo
            pos = cursor[pl.ds(b, L)][0]
            plsc.store_compressed(
                cursor.at[pl.ds(b, L)],
                jnp.broadcast_to(pos + 1, (L,)),
                mask=lane0,
            )
            plsc.store_compressed(
                ws_i.at[pl.ds(pos, L)], jnp.broadcast_to(iq, (L,)), mask=lane0
            )
            plsc.store_compressed(
                ws_j.at[pl.ds(pos, L)], jnp.broadcast_to(jq, (L,)), mask=lane0
            )
            return 0

        lax.fori_loop(0, n_wl, place_body, 0)

        def fire(kk, slot):
            cc = pl.multiple_of((c_lo + kk) * 128, 128)
            pltpu.async_copy(
                tableT_hbm.at[:, pl.ds(cc, 128)], chunks[slot], sems[slot]
            )

        def wait(slot):
            pltpu.make_async_copy(
                tableT_hbm.at[:, pl.ds(0, 128)], chunks[slot], sems[slot]
            ).wait()

        for p in range(_DEPTH - 1):

            @pl.when(p < nc_w)
            def _():
                fire(p, p)

        def sweep_body(it, r):
            for m in range(_DEPTH):
                kk = it * _DEPTH + m
                valid = kk < nc_w

                @pl.when((kk + _DEPTH - 1) < nc_w)
                def _():
                    fire(kk + _DEPTH - 1, (m + _DEPTH - 1) % _DEPTH)

                @pl.when(valid)
                def _():
                    wait(m)

                lo = starts0[pl.ds(jnp.minimum(kk, n_hist - 1), L)][0]
                hi = cursor[pl.ds(jnp.minimum(kk, n_hist - 1), L)][0]
                lo = jnp.where(valid, lo, 0)
                hi = jnp.where(valid, hi, 0)

                def extract_body(q, rr):
                    @pl.when(rr >= _ROWS)
                    def _():
                        pltpu.make_async_copy(
                            rows_v.at[0], out_hbm.at[0], sem_out
                        ).wait()

                    slot_r = rr & (_ROWS - 1)
                    iq = ws_i[pl.ds(q, L)][0]
                    jq = ws_j[pl.ds(q, L)][0]
                    lane = jnp.broadcast_to(iq & 127, (L,))
                    for g2 in range(D // L):
                        v = plsc.load_gather(
                            chunks[m], [iota + L * g2, lane]
                        )
                        rows_v[slot_r, pl.ds(L * g2, L)] = v
                    pltpu.async_copy(
                        rows_v.at[slot_r], out_hbm.at[jq], sem_out
                    )
                    return rr + 1

                r = lax.fori_loop(lo, hi, extract_body, r)
            return r

        n_it = (cpw + _DEPTH - 1) // _DEPTH
        r_total = lax.fori_loop(0, n_it, sweep_body, jnp.int32(0))

        def drain_body(q, _):
            pltpu.make_async_copy(
                rows_v.at[0], out_hbm.at[0], sem_out
            ).wait()
            return 0

        lax.fori_loop(0, jnp.minimum(r_total, _ROWS), drain_body, 0)

    return k


def kernel(indices, table):
    idx = indices.astype(jnp.int32)
    (B,) = idx.shape
    V, D = table.shape
    return _make_gather(V, D, B)(idx, table.T)


# speedup vs baseline: 4.0606x; 1.0006x over previous
import functools

import jax
import jax.numpy as jnp
from jax import lax
from jax.experimental import pallas as pl
from jax.experimental.pallas import tpu as pltpu
from jax.experimental.pallas import tpu_sc as plsc

_DEPTH = 8
_WLMAX = 1024
_ROWS = 256


@functools.lru_cache(maxsize=None)
def _make_gather(V, D, B):
    info = plsc.get_sparse_core_info()
    NC, NS, L = info.num_cores, info.num_subcores, info.num_lanes
    NW = NC * NS
    assert B % (16 * NW) == 0 and D % L == 0
    n_chunk = (V + 127) // 128
    cpw = (n_chunk + NW - 1) // NW
    n_hist = ((cpw + 15) // 16) * 16
    mesh = plsc.VectorSubcoreMesh(core_axis_name="c", subcore_axis_name="s")

    @functools.partial(
        pl.kernel,
        mesh=mesh,
        compiler_params=pltpu.CompilerParams(needs_layout_passes=False),
        out_type=jax.ShapeDtypeStruct((B, D), jnp.float32),
        scratch_types=[
            pltpu.VMEM((B,), jnp.int32),
            pltpu.VMEM((_WLMAX + 16,), jnp.int32),
            pltpu.VMEM((_WLMAX + 16,), jnp.int32),
            pltpu.VMEM((_WLMAX + 16,), jnp.int32),
            pltpu.VMEM((_WLMAX + 16,), jnp.int32),
            pltpu.VMEM((n_hist + 16,), jnp.int32),
            pltpu.VMEM((n_hist + 16,), jnp.int32),
            pltpu.VMEM((n_hist + 16,), jnp.int32),
            *[pltpu.VMEM((D, 128), jnp.float32) for _ in range(_DEPTH)],
            pltpu.VMEM((_ROWS, D), jnp.float32),
            pltpu.SemaphoreType.DMA,
            *[pltpu.SemaphoreType.DMA for _ in range(_DEPTH)],
        ],
    )
    def k(idx_hbm, tableT_hbm, out_hbm, idx_v, wl_i, wl_j, ws_i, ws_j,
          hist, starts0, cursor, *rest):
        chunks = rest[:_DEPTH]
        rows_v = rest[_DEPTH]
        sem_out = rest[_DEPTH + 1]
        sems = rest[_DEPTH + 2 :]
        wid = lax.axis_index("s") * NC + lax.axis_index("c")
        c_lo = wid * cpw
        c_hi = jnp.minimum(c_lo + cpw, n_chunk)
        nc_w = c_hi - c_lo

        iota = lax.iota(jnp.int32, L)
        lane0 = iota == 0
        zeros = jnp.zeros((L,), jnp.int32)

        def fire(kk, slot):
            cc = pl.multiple_of((c_lo + kk) * 128, 128)
            pltpu.async_copy(
                tableT_hbm.at[:, pl.ds(cc, 128)], chunks[slot], sems[slot]
            )

        def wait(slot):
            pltpu.make_async_copy(
                tableT_hbm.at[:, pl.ds(0, 128)], chunks[slot], sems[slot]
            ).wait()

        for p in range(_DEPTH - 1):

            @pl.when(p < nc_w)
            def _():
                fire(p, p)

        pltpu.sync_copy(idx_hbm, idx_v)

        def scan_body(n, off):
            vec = idx_v[pl.ds(n * L, L)]
            cvec = vec >> 7
            mask = (cvec >= c_lo) & (cvec < c_hi)
            plsc.store_compressed(wl_i.at[pl.ds(off, L)], vec, mask=mask)
            plsc.store_compressed(
                wl_j.at[pl.ds(off, L)], n * L + iota, mask=mask
            )
            return off + plsc.all_reduce_population_count(mask)[0]

        n_wl = lax.fori_loop(0, B // L, scan_body, jnp.int32(0), unroll=4)

        for gp in range(n_hist // 16):
            hist[pl.ds(gp * 16, 16)] = zeros

        def count_body(q, _):
            iq = wl_i[pl.ds(q, L)][0]
            b = (iq >> 7) - c_lo
            cnt = hist[pl.ds(b, L)]
            plsc.store_compressed(hist.at[pl.ds(b, L)], cnt + 1, mask=lane0)
            return 0

        lax.fori_loop(0, n_wl, count_body, 0)

        carry = jnp.int32(0)
        for gp in range(n_hist // 16):
            hv = hist[pl.ds(gp * 16, 16)]
            cs = plsc.cumsum(hv)
            sv = cs - hv + carry
            starts0[pl.ds(gp * 16, 16)] = sv
            cursor[pl.ds(gp * 16, 16)] = sv
            carry = sv[15] + hv[15]

        def place_body(q, _):
            iq = wl_i[pl.ds(q, L)][0]
            jq = wl_j[pl.ds(q, L)][0]
            b = (iq >> 7) - c_lo
            pos = cursor[pl.ds(b, L)][0]
            plsc.store_compressed(
                cursor.at[pl.ds(b, L)],
                jnp.broadcast_to(pos + 1, (L,)),
                mask=lane0,
            )
            plsc.store_compressed(
                ws_i.at[pl.ds(pos, L)], jnp.broadcast_to(iq, (L,)), mask=lane0
            )
            plsc.store_compressed(
                ws_j.at[pl.ds(pos, L)], jnp.broadcast_to(jq, (L,)), mask=lane0
            )
            return 0

        lax.fori_loop(0, n_wl, place_body, 0)

        def sweep_body(it, r):
            for m in range(_DEPTH):
                kk = it * _DEPTH + m
                valid = kk < nc_w

                @pl.when((kk + _DEPTH - 1) < nc_w)
                def _():
                    fire(kk + _DEPTH - 1, (m + _DEPTH - 1) % _DEPTH)

                @pl.when(valid)
                def _():
                    wait(m)

                lo = starts0[pl.ds(jnp.minimum(kk, n_hist - 1), L)][0]
                hi = cursor[pl.ds(jnp.minimum(kk, n_hist - 1), L)][0]
                lo = jnp.where(valid, lo, 0)
                hi = jnp.where(valid, hi, 0)

                def extract_body(q, rr):
                    @pl.when(rr >= _ROWS)
                    def _():
                        pltpu.make_async_copy(
                            rows_v.at[0], out_hbm.at[0], sem_out
                        ).wait()

                    slot_r = rr & (_ROWS - 1)
                    iq = ws_i[pl.ds(q, L)][0]
                    jq = ws_j[pl.ds(q, L)][0]
                    lane = jnp.broadcast_to(iq & 127, (L,))
                    for g2 in range(D // L):
                        v = plsc.load_gather(
                            chunks[m], [iota + L * g2, lane]
                        )
                        rows_v[slot_r, pl.ds(L * g2, L)] = v
                    pltpu.async_copy(
                        rows_v.at[slot_r], out_hbm.at[jq], sem_out
                    )
                    return rr + 1

                r = lax.fori_loop(lo, hi, extract_body, r)
            return r

        n_it = (cpw + _DEPTH - 1) // _DEPTH
        r_total = lax.fori_loop(0, n_it, sweep_body, jnp.int32(0))

        def drain_body(q, _):
            pltpu.make_async_copy(
                rows_v.at[0], out_hbm.at[0], sem_out
            ).wait()
            return 0

        lax.fori_loop(0, jnp.minimum(r_total, _ROWS), drain_body, 0)

    return k


def kernel(indices, table):
    idx = indices.astype(jnp.int32)
    (B,) = idx.shape
    V, D = table.shape
    return _make_gather(V, D, B)(idx, table.T)


# speedup vs baseline: 4.3329x; 1.0670x over previous
import functools

import jax
import jax.numpy as jnp
from jax import lax
from jax.experimental import pallas as pl
from jax.experimental.pallas import tpu as pltpu
from jax.experimental.pallas import tpu_sc as plsc

_DEPTH = 8
_WLMAX = 1024
_ROWS = 256


@functools.lru_cache(maxsize=None)
def _make_gather(V, D, B):
    info = plsc.get_sparse_core_info()
    NC, NS, L = info.num_cores, info.num_subcores, info.num_lanes
    NW = NC * NS
    assert B % (16 * NW) == 0 and D % L == 0
    n_chunk = (V + 127) // 128
    cpw = (n_chunk + NW - 1) // NW
    n_hist = ((cpw + 15) // 16) * 16
    mesh = plsc.VectorSubcoreMesh(core_axis_name="c", subcore_axis_name="s")

    @functools.partial(
        pl.kernel,
        mesh=mesh,
        compiler_params=pltpu.CompilerParams(needs_layout_passes=False),
        out_type=jax.ShapeDtypeStruct((B, D), jnp.float32),
        scratch_types=[
            pltpu.VMEM((B,), jnp.int32),
            pltpu.VMEM((_WLMAX + 16,), jnp.int32),
            pltpu.VMEM((_WLMAX + 16,), jnp.int32),
            pltpu.VMEM((_WLMAX + 16,), jnp.int32),
            pltpu.VMEM((_WLMAX + 16,), jnp.int32),
            pltpu.VMEM((n_hist + 16,), jnp.int32),
            pltpu.VMEM((n_hist + 16,), jnp.int32),
            pltpu.VMEM((n_hist + 16,), jnp.int32),
            pltpu.VMEM((n_hist + 16,), jnp.int32),
            pltpu.VMEM((n_hist + 16,), jnp.int32),
            pltpu.VMEM((n_hist + 16,), jnp.int32),
            *[pltpu.VMEM((D, 128), jnp.float32) for _ in range(_DEPTH)],
            pltpu.VMEM((_ROWS, D), jnp.float32),
            pltpu.SemaphoreType.DMA,
            *[pltpu.SemaphoreType.DMA for _ in range(_DEPTH)],
        ],
    )
    def k(idx_hbm, tableT_hbm, out_hbm, idx_v, wl_i, wl_j, ws_i, ws_j,
          hist, starts0, cursor, cl_c, cl_lo, cl_hi, *rest):
        chunks = rest[:_DEPTH]
        rows_v = rest[_DEPTH]
        sem_out = rest[_DEPTH + 1]
        sems = rest[_DEPTH + 2 :]
        wid = lax.axis_index("s") * NC + lax.axis_index("c")
        c_lo = wid * cpw
        c_hi = jnp.minimum(c_lo + cpw, n_chunk)
        nc_w = c_hi - c_lo

        iota = lax.iota(jnp.int32, L)
        lane0 = iota == 0
        zeros = jnp.zeros((L,), jnp.int32)

        def fire(kk, slot):
            cid = cl_c[pl.ds(kk, L)][0]
            cc = pl.multiple_of(cid * 128, 128)
            pltpu.async_copy(
                tableT_hbm.at[:, pl.ds(cc, 128)], chunks[slot], sems[slot]
            )

        def wait(slot):
            pltpu.make_async_copy(
                tableT_hbm.at[:, pl.ds(0, 128)], chunks[slot], sems[slot]
            ).wait()

        pltpu.sync_copy(idx_hbm, idx_v)

        def scan_body(n, off):
            vec = idx_v[pl.ds(n * L, L)]
            cvec = vec >> 7
            mask = (cvec >= c_lo) & (cvec < c_hi)
            plsc.store_compressed(wl_i.at[pl.ds(off, L)], vec, mask=mask)
            plsc.store_compressed(
                wl_j.at[pl.ds(off, L)], n * L + iota, mask=mask
            )
            return off + plsc.all_reduce_population_count(mask)[0]

        n_wl = lax.fori_loop(0, B // L, scan_body, jnp.int32(0), unroll=4)

        for gp in range(n_hist // 16):
            hist[pl.ds(gp * 16, 16)] = zeros

        def count_body(q, _):
            iq = wl_i[pl.ds(q, L)][0]
            b = (iq >> 7) - c_lo
            cnt = hist[pl.ds(b, L)]
            plsc.store_compressed(hist.at[pl.ds(b, L)], cnt + 1, mask=lane0)
            return 0

        lax.fori_loop(0, n_wl, count_body, 0)

        carry = jnp.int32(0)
        for gp in range(n_hist // 16):
            hv = hist[pl.ds(gp * 16, 16)]
            cs = plsc.cumsum(hv)
            sv = cs - hv + carry
            starts0[pl.ds(gp * 16, 16)] = sv
            cursor[pl.ds(gp * 16, 16)] = sv
            carry = sv[15] + hv[15]

        def place_body(q, _):
            iq = wl_i[pl.ds(q, L)][0]
            jq = wl_j[pl.ds(q, L)][0]
            b = (iq >> 7) - c_lo
            pos = cursor[pl.ds(b, L)][0]
            plsc.store_compressed(
                cursor.at[pl.ds(b, L)],
                jnp.broadcast_to(pos + 1, (L,)),
                mask=lane0,
            )
            plsc.store_compressed(
                ws_i.at[pl.ds(pos, L)], jnp.broadcast_to(iq, (L,)), mask=lane0
            )
            plsc.store_compressed(
                ws_j.at[pl.ds(pos, L)], jnp.broadcast_to(jq, (L,)), mask=lane0
            )
            return 0

        lax.fori_loop(0, n_wl, place_body, 0)

        n_ne = jnp.int32(0)
        for gp in range(n_hist // 16):
            hv = hist[pl.ds(gp * 16, 16)]
            mask = hv > 0
            plsc.store_compressed(
                cl_c.at[pl.ds(n_ne, L)], c_lo + gp * 16 + iota, mask=mask
            )
            plsc.store_compressed(
                cl_lo.at[pl.ds(n_ne, L)],
                starts0[pl.ds(gp * 16, 16)],
                mask=mask,
            )
            plsc.store_compressed(
                cl_hi.at[pl.ds(n_ne, L)],
                cursor[pl.ds(gp * 16, 16)],
                mask=mask,
            )
            n_ne = n_ne + plsc.all_reduce_population_count(mask)[0]

        for p in range(_DEPTH - 1):

            @pl.when(p < n_ne)
            def _():
                fire(p, p)

        def sweep_body(it, r):
            for m in range(_DEPTH):
                kk = it * _DEPTH + m
                valid = kk < n_ne

                @pl.when((kk + _DEPTH - 1) < n_ne)
                def _():
                    fire(kk + _DEPTH - 1, (m + _DEPTH - 1) % _DEPTH)

                @pl.when(valid)
                def _():
                    wait(m)

                lo = cl_lo[pl.ds(kk, L)][0]
                hi = cl_hi[pl.ds(kk, L)][0]
                lo = jnp.where(valid, lo, 0)
                hi = jnp.where(valid, hi, 0)

                def extract_body(q, rr):
                    @pl.when(rr >= _ROWS)
                    def _():
                        pltpu.make_async_copy(
                            rows_v.at[0], out_hbm.at[0], sem_out
                        ).wait()

                    slot_r = rr & (_ROWS - 1)
                    iq = ws_i[pl.ds(q, L)][0]
                    jq = ws_j[pl.ds(q, L)][0]
                    lane = jnp.broadcast_to(iq & 127, (L,))
                    for g2 in range(D // L):
                        v = plsc.load_gather(
                            chunks[m], [iota + L * g2, lane]
                        )
                        rows_v[slot_r, pl.ds(L * g2, L)] = v
                    pltpu.async_copy(
                        rows_v.at[slot_r], out_hbm.at[jq], sem_out
                    )
                    return rr + 1

                r = lax.fori_loop(lo, hi, extract_body, r)
            return r

        n_it = (cpw + _DEPTH - 1) // _DEPTH
        r_total = lax.fori_loop(0, n_it, sweep_body, jnp.int32(0))

        def drain_body(q, _):
            pltpu.make_async_copy(
                rows_v.at[0], out_hbm.at[0], sem_out
            ).wait()
            return 0

        lax.fori_loop(0, jnp.minimum(r_total, _ROWS), drain_body, 0)

    return k


def kernel(indices, table):
    idx = indices.astype(jnp.int32)
    (B,) = idx.shape
    V, D = table.shape
    return _make_gather(V, D, B)(idx, table.T)


# speedup vs baseline: 4.3946x; 1.0142x over previous
import functools

import jax
import jax.numpy as jnp
from jax import lax
from jax.experimental import pallas as pl
from jax.experimental.pallas import tpu as pltpu
from jax.experimental.pallas import tpu_sc as plsc

_DEPTH = 8
_ROWS = 64


@functools.lru_cache(maxsize=None)
def _make_gather(V, D, B):
    info = plsc.get_sparse_core_info()
    NC, NS, L = info.num_cores, info.num_subcores, info.num_lanes
    NW = NC * NS
    assert B % (16 * NW) == 0 and D % L == 0
    assert B & (B - 1) == 0 and B <= (1 << 14)
    n_chunk = (V + 127) // 128
    cpw = (n_chunk + NW - 1) // NW
    n_hist = ((cpw + 15) // 16) * 16
    mesh = plsc.VectorSubcoreMesh(core_axis_name="c", subcore_axis_name="s")

    @functools.partial(
        pl.kernel,
        mesh=mesh,
        compiler_params=pltpu.CompilerParams(needs_layout_passes=False),
        out_type=jax.ShapeDtypeStruct((B, D), jnp.float32),
        scratch_types=[
            pltpu.VMEM((B,), jnp.int32),
            pltpu.VMEM((B + 16,), jnp.int32),
            pltpu.VMEM((B + 16,), jnp.int32),
            pltpu.VMEM((n_hist + 16,), jnp.int32),
            pltpu.VMEM((n_hist + 16,), jnp.int32),
            pltpu.VMEM((n_hist + 16,), jnp.int32),
            pltpu.VMEM((n_hist + 16,), jnp.int32),
            pltpu.VMEM((n_hist + 16,), jnp.int32),
            pltpu.VMEM((n_hist + 16,), jnp.int32),
            *[pltpu.VMEM((D, 128), jnp.float32) for _ in range(_DEPTH)],
            pltpu.VMEM((_ROWS, D), jnp.float32),
            pltpu.SemaphoreType.DMA,
            *[pltpu.SemaphoreType.DMA for _ in range(_DEPTH)],
        ],
    )
    def k(idx_hbm, tableT_hbm, out_hbm, idx_v, wl_p, ws_p,
          hist, starts0, cursor, cl_c, cl_lo, cl_hi, *rest):
        chunks = rest[:_DEPTH]
        rows_v = rest[_DEPTH]
        sem_out = rest[_DEPTH + 1]
        sems = rest[_DEPTH + 2 :]
        wid = lax.axis_index("s") * NC + lax.axis_index("c")
        c_lo = wid * cpw
        c_hi = jnp.minimum(c_lo + cpw, n_chunk)
        nc_w = c_hi - c_lo

        iota = lax.iota(jnp.int32, L)
        lane0 = iota == 0
        zeros = jnp.zeros((L,), jnp.int32)

        def fire(kk, slot):
            cid = cl_c[pl.ds(kk, L)][0]
            cc = pl.multiple_of(cid * 128, 128)
            pltpu.async_copy(
                tableT_hbm.at[:, pl.ds(cc, 128)], chunks[slot], sems[slot]
            )

        def wait(slot):
            pltpu.make_async_copy(
                tableT_hbm.at[:, pl.ds(0, 128)], chunks[slot], sems[slot]
            ).wait()

        pltpu.sync_copy(idx_hbm, idx_v)

        def scan_body(n, off):
            vec = idx_v[pl.ds(n * L, L)]
            cvec = vec >> 7
            mask = (cvec >= c_lo) & (cvec < c_hi)
            packed = ((cvec - c_lo) << 21) | ((n * L + iota) << 7) | (vec & 127)
            plsc.store_compressed(wl_p.at[pl.ds(off, L)], packed, mask=mask)
            return off + plsc.all_reduce_population_count(mask)[0]

        n_wl = lax.fori_loop(0, B // L, scan_body, jnp.int32(0), unroll=4)

        for gp in range(n_hist // 16):
            hist[pl.ds(gp * 16, 16)] = zeros

        def count_body(q, _):
            b = wl_p[pl.ds(q, L)][0] >> 21
            cnt = hist[pl.ds(b, L)]
            plsc.store_compressed(hist.at[pl.ds(b, L)], cnt + 1, mask=lane0)
            return 0

        lax.fori_loop(0, n_wl, count_body, 0)

        carry = jnp.int32(0)
        for gp in range(n_hist // 16):
            hv = hist[pl.ds(gp * 16, 16)]
            cs = plsc.cumsum(hv)
            sv = cs - hv + carry
            starts0[pl.ds(gp * 16, 16)] = sv
            cursor[pl.ds(gp * 16, 16)] = sv
            carry = sv[15] + hv[15]

        def place_body(q, _):
            e = wl_p[pl.ds(q, L)][0]
            b = e >> 21
            pos = cursor[pl.ds(b, L)][0]
            plsc.store_compressed(
                cursor.at[pl.ds(b, L)],
                jnp.broadcast_to(pos + 1, (L,)),
                mask=lane0,
            )
            plsc.store_compressed(
                ws_p.at[pl.ds(pos, L)], jnp.broadcast_to(e, (L,)), mask=lane0
            )
            return 0

        lax.fori_loop(0, n_wl, place_body, 0)

        n_ne = jnp.int32(0)
        for gp in range(n_hist // 16):
            hv = hist[pl.ds(gp * 16, 16)]
            mask = hv > 0
            plsc.store_compressed(
                cl_c.at[pl.ds(n_ne, L)], c_lo + gp * 16 + iota, mask=mask
            )
            plsc.store_compressed(
                cl_lo.at[pl.ds(n_ne, L)],
                starts0[pl.ds(gp * 16, 16)],
                mask=mask,
            )
            plsc.store_compressed(
                cl_hi.at[pl.ds(n_ne, L)],
                cursor[pl.ds(gp * 16, 16)],
                mask=mask,
            )
            n_ne = n_ne + plsc.all_reduce_population_count(mask)[0]

        for p in range(_DEPTH - 1):

            @pl.when(p < n_ne)
            def _():
                fire(p, p)

        def sweep_body(it, r):
            for m in range(_DEPTH):
                kk = it * _DEPTH + m
                valid = kk < n_ne

                @pl.when((kk + _DEPTH - 1) < n_ne)
                def _():
                    fire(kk + _DEPTH - 1, (m + _DEPTH - 1) % _DEPTH)

                @pl.when(valid)
                def _():
                    wait(m)

                lo = cl_lo[pl.ds(kk, L)][0]
                hi = cl_hi[pl.ds(kk, L)][0]
                lo = jnp.where(valid, lo, 0)
                hi = jnp.where(valid, hi, 0)

                def extract_body(q, rr):
                    @pl.when(rr >= _ROWS)
                    def _():
                        pltpu.make_async_copy(
                            rows_v.at[0], out_hbm.at[0], sem_out
                        ).wait()

                    slot_r = rr & (_ROWS - 1)
                    e = ws_p[pl.ds(q, L)][0]
                    jq = (e >> 7) & (B - 1)
                    lane = jnp.broadcast_to(e & 127, (L,))
                    for g2 in range(D // L):
                        v = plsc.load_gather(
                            chunks[m], [iota + L * g2, lane]
                        )
                        rows_v[slot_r, pl.ds(L * g2, L)] = v
                    pltpu.async_copy(
                        rows_v.at[slot_r], out_hbm.at[jq], sem_out
                    )
                    return rr + 1

                r = lax.fori_loop(lo, hi, extract_body, r)
            return r

        n_it = (cpw + _DEPTH - 1) // _DEPTH
        r_total = lax.fori_loop(0, n_it, sweep_body, jnp.int32(0))

        def drain_body(q, _):
            pltpu.make_async_copy(
                rows_v.at[0], out_hbm.at[0], sem_out
            ).wait()
            return 0

        lax.fori_loop(0, jnp.minimum(r_total, _ROWS), drain_body, 0)

    return k


def kernel(indices, table):
    idx = indices.astype(jnp.int32)
    (B,) = idx.shape
    V, D = table.shape
    return _make_gather(V, D, B)(idx, table.T)
